# baseline (speedup 1.0000x reference)
import math

import jax
import jax.numpy as jnp
from jax import lax
from jax.experimental import pallas as pl
from jax.experimental.pallas import tpu as pltpu
from jax.experimental.pallas import tpu_sc as plsc

N = 10000
E = 320000
NHID = 64

NC = 2
NS = 16
NW = NC * NS
EPT = E // NW
CH = 80
NITER = EPT // CH
RPT = 632
NP = NS * RPT

def _dot(a, b):
  return jax.lax.dot(a, b)


def _elu(v):
  return jnp.where(v > 0, v, jnp.exp(jnp.minimum(v, 0.0)) - 1.0)


def _rownorm(v):
  ssq = _dot(v * v, jnp.ones((v.shape[1], 1), jnp.float32))
  return v * jax.lax.rsqrt(jnp.maximum(ssq, 1e-24))


ZR = RPT // 8
NBUF = 6
D = NHID


_SC_MESH = plsc.VectorSubcoreMesh(
    core_axis_name="c", subcore_axis_name="s", num_cores=NC, num_subcores=NS)
_SC_PARAMS = pltpu.CompilerParams(
    use_tc_tiling_on_sc=False, needs_layout_passes=False)


def _make_sc_agg():

  def body(table, adj2d, out, srcv, dstv, rows, zbuf, acc, gsem, ssem):
    c = lax.axis_index("c")
    s = lax.axis_index("s")
    wid = c * NS + s

    zeros = jnp.zeros((16,), jnp.float32)

    def zinit(r, carry):
      for k in range(D // 16):
        zbuf[r, pl.ds(k * 16, 16)] = zeros
      return carry

    lax.fori_loop(0, ZR, zinit, 0)
    for z in range(RPT // ZR):
      pltpu.sync_copy(zbuf, acc.at[pl.ds(s * RPT + z * ZR, ZR)])
    plsc.subcore_barrier()

    pltpu.sync_copy(adj2d.at[pl.ds(wid * NITER, NITER)], srcv)
    pltpu.sync_copy(adj2d.at[pl.ds(E // CH + wid * NITER, NITER)], dstv)

    for j in range(NBUF - 1):
      pltpu.async_copy(table.at[srcv.at[j]], rows.at[j], gsem.at[j])

    def eloop(i, carry):
      b = lax.rem(i, NBUF)
      pb = lax.rem(i + NBUF - 1, NBUF)
      g = i + NBUF - 1

      @pl.when(jnp.logical_and(i >= 1, g < NITER))
      def _():
        pltpu.make_async_copy(
            rows.at[pb], acc.at[dstv.at[i - 1]], ssem.at[pb]).wait()

      @pl.when(g < NITER)
      def _():
        pltpu.async_copy(table.at[srcv.at[g]], rows.at[pb], gsem.at[pb])

      pltpu.make_async_copy(table.at[srcv.at[i]], rows.at[b], gsem.at[b]).wait()
      pltpu.async_copy(rows.at[b], acc.at[dstv.at[i]], ssem.at[b], add=True)
      return carry

    lax.fori_loop(0, NITER, eloop, 0)
    for k in range(NBUF):
      ci = NITER - NBUF + k
      pltpu.make_async_copy(
          rows.at[ci % NBUF], acc.at[dstv.at[ci]], ssem.at[ci % NBUF]).wait()
    plsc.subcore_barrier()

    pltpu.sync_copy(acc.at[pl.ds(s * RPT, RPT)], out.at[c, pl.ds(s * RPT, RPT)])

  return pl.kernel(
      body,
      out_type=[jax.ShapeDtypeStruct((NC, NP, D), jnp.float32)],
      mesh=_SC_MESH,
      scratch_types=[
          pltpu.VMEM((NITER, CH), jnp.int32),
          pltpu.VMEM((NITER, CH), jnp.int32),
          pltpu.VMEM((NBUF, CH, D), jnp.float32),
          pltpu.VMEM((ZR, D), jnp.float32),
          pltpu.VMEM_SHARED((NP, D), jnp.float32),
          pltpu.SemaphoreType.DMA((NBUF,)),
          pltpu.SemaphoreType.DMA((NBUF,)),
      ],
      compiler_params=_SC_PARAMS,
  )


def _make_sc_deg():

  def body(adj2d, dout, dstv, degp):
    c = lax.axis_index("c")
    s = lax.axis_index("s")
    wid = c * NS + s
    pltpu.sync_copy(adj2d.at[pl.ds(E // CH + wid * NITER, NITER)], dstv)

    zeros = jnp.zeros((16,), jnp.float32)

    def dzinit(r, carry):
      degp[pl.ds(r * 16, 16)] = zeros
      return carry

    lax.fori_loop(0, NP // 16, dzinit, 0)

    ones = jnp.full((16,), 1.0, jnp.float32)

    def dloop(i, carry):
      for k in range(CH // 16):
        plsc.addupdate_scatter(degp, [dstv[i, pl.ds(k * 16, 16)]], ones)
      return carry

    lax.fori_loop(0, NITER, dloop, 0)
    pltpu.sync_copy(degp, dout.at[wid])

  return pl.kernel(
      body,
      out_type=[jax.ShapeDtypeStruct((NW, NP), jnp.float32)],
      mesh=_SC_MESH,
      scratch_types=[
          pltpu.VMEM((NITER, CH), jnp.int32),
          pltpu.VMEM((NP,), jnp.float32),
      ],
      compiler_params=_SC_PARAMS,
  )


_sc_agg = _make_sc_agg()
_sc_deg = _make_sc_deg()


RB = 2560
GRID = (N + RB - 1) // RB


def _ka1_body(x_ref, q_ref, Wl0x, Wl0q, T0_ref):
  T0_ref[...] = _dot(x_ref[...], Wl0x[...]) + _dot(q_ref[...], Wl0q[...])


def _ka2_body(t_ref, x_ref, q_ref, freq_ref, Wt1, bt1, Wt2, bt2,
              Wr0x, Wr0q, R0_ref, temb_ref):
  emb = (t_ref[...] * 4.0) * freq_ref[...]
  temb0 = jnp.concatenate([jnp.sin(emb), jnp.cos(emb)], axis=1)
  hmid = _elu(_dot(temb0, Wt1[...]) + bt1[...])
  temb_ref[...] = _dot(hmid, Wt2[...]) + bt2[...]
  R0_ref[...] = _dot(x_ref[...], Wr0x[...]) + _dot(q_ref[...], Wr0q[...])


def _kb_body(agg_ref, deg_ref, R0_ref, temb_ref, q_ref, bl0,
             Wl1x, Wl1q, Wr1x, Wr1q, T1_ref, R1_ref, rdeg_ref):
  a = agg_ref[0] + agg_ref[1]
  deg = jnp.sum(deg_ref[...], axis=0)
  rdeg = (1.0 / jnp.maximum(deg, 1.0)).reshape(RB, 1)
  out0 = a * rdeg + bl0[...] + R0_ref[...]
  h = jnp.maximum(_rownorm(out0) + temb_ref[...], 0.0)
  q = q_ref[...]
  T1_ref[...] = _dot(h, Wl1x[...]) + _dot(q, Wl1q[...])
  R1_ref[...] = _dot(h, Wr1x[...]) + _dot(q, Wr1q[...])
  rdeg_ref[...] = rdeg


def _kc_body(agg_ref, rdeg_ref, R1_ref, temb_ref, q_ref, bl1,
             Wf1h, Wf1q, bf1, Wf2, bf2, out_ref):
  a = agg_ref[0] + agg_ref[1]
  out1 = a * rdeg_ref[...] + bl1[...] + R1_ref[...]
  h = jnp.maximum(_rownorm(out1) + temb_ref[...], 0.0)
  q = q_ref[...]
  f = _elu(_dot(h, Wf1h[...]) + _dot(q, Wf1q[...]) + bf1[...])
  out_ref[...] = _dot(f, Wf2[...]) + bf2[...]


def _row_spec(d):
  return pl.BlockSpec((RB, d), lambda i: (i, 0))


def _full_spec(shape):
  nd = len(shape)
  return pl.BlockSpec(shape, lambda i, _n=nd: (0,) * _n)


def _agg_spec(d):
  return pl.BlockSpec((NC, RB, d), lambda i: (0, i, 0))


def kernel(x, q_Y_sample, adj, t, num_steps, W_t1, b_t1, W_t2, b_t2,
           Wl0, bl0, Wr0, Wl1, bl1, Wr1, Wf1, bf1, Wf2, bf2):
  del num_steps
  adj2d = adj.astype(jnp.int32).reshape(2 * (E // CH), CH)
  f32 = jnp.float32
  half = NHID // 2
  freq = jnp.exp(
      jnp.arange(half, dtype=f32) * (-math.log(10000.0) / (half - 1)))
  freq = freq.reshape(1, half)

  nfeat = x.shape[1]
  q = q_Y_sample

  ka1 = pl.pallas_call(
      _ka1_body,
      grid=(GRID,),
      in_specs=[
          _row_spec(nfeat), _row_spec(q.shape[1]),
          _full_spec((nfeat, NHID)), _full_spec((q.shape[1], NHID)),
      ],
      out_specs=[_row_spec(NHID)],
      out_shape=[jax.ShapeDtypeStruct((N, NHID), f32)],
  )
  (degp,) = _sc_deg(adj2d)
  (T0,) = ka1(x, q, Wl0[:nfeat], Wl0[nfeat:])

  (agg0,) = _sc_agg(T0, adj2d)

  ka2 = pl.pallas_call(
      _ka2_body,
      grid=(GRID,),
      in_specs=[
          _row_spec(1), _row_spec(nfeat), _row_spec(q.shape[1]),
          _full_spec(freq.shape),
          _full_spec(W_t1.shape), _full_spec((1, b_t1.shape[0])),
          _full_spec(W_t2.shape), _full_spec((1, b_t2.shape[0])),
          _full_spec((nfeat, NHID)), _full_spec((q.shape[1], NHID)),
      ],
      out_specs=[_row_spec(NHID), _row_spec(NHID)],
      out_shape=[
          jax.ShapeDtypeStruct((N, NHID), f32),
          jax.ShapeDtypeStruct((N, NHID), f32),
      ],
  )
  R0, temb = ka2(
      t.reshape(N, 1), x, q, freq,
      W_t1, b_t1.reshape(1, -1), W_t2, b_t2.reshape(1, -1),
      Wr0[:nfeat], Wr0[nfeat:])

  kb = pl.pallas_call(
      _kb_body,
      grid=(GRID,),
      in_specs=[
          _agg_spec(NHID), pl.BlockSpec((NW, RB), lambda i: (0, i)),
          _row_spec(NHID), _row_spec(NHID),
          _row_spec(q.shape[1]), _full_spec((1, NHID)),
          _full_spec((NHID, NHID)), _full_spec((q.shape[1], NHID)),
          _full_spec((NHID, NHID)), _full_spec((q.shape[1], NHID)),
      ],
      out_specs=[_row_spec(NHID), _row_spec(NHID), _row_spec(1)],
      out_shape=[
          jax.ShapeDtypeStruct((N, NHID), f32),
          jax.ShapeDtypeStruct((N, NHID), f32),
          jax.ShapeDtypeStruct((N, 1), f32),
      ],
  )
  T1, R1, rdeg = kb(
      agg0, degp, R0, temb, q, bl0.reshape(1, -1),
      Wl1[:NHID], Wl1[NHID:], Wr1[:NHID], Wr1[NHID:])

  agg1, = _sc_agg(T1, adj2d)

  nout = bf2.shape[0]
  f1 = bf1.shape[0]
  kc = pl.pallas_call(
      _kc_body,
      grid=(GRID,),
      in_specs=[
          _agg_spec(NHID), _row_spec(1), _row_spec(NHID), _row_spec(NHID),
          _row_spec(q.shape[1]), _full_spec((1, NHID)),
          _full_spec((NHID, f1)), _full_spec((q.shape[1], f1)),
          _full_spec((1, f1)), _full_spec((f1, nout)), _full_spec((1, nout)),
      ],
      out_specs=[_row_spec(nout)],
      out_shape=[jax.ShapeDtypeStruct((N, nout), f32)],
  )
  (out,) = kc(
      agg1, rdeg, R1, temb, q, bl1.reshape(1, -1),
      Wf1[:NHID], Wf1[NHID:], bf1.reshape(1, -1), Wf2, bf2.reshape(1, -1))
  return out


# speedup vs baseline: 1.0041x; 1.0041x over previous
import math

import jax
import jax.numpy as jnp
from jax import lax
from jax.experimental import pallas as pl
from jax.experimental.pallas import tpu as pltpu
from jax.experimental.pallas import tpu_sc as plsc

N = 10000
E = 320000
NHID = 64

NC = 2
NS = 16
NW = NC * NS
EPT = E // NW
CH = 80
NITER = EPT // CH
RPT = 632
NP = NS * RPT

def _dot(a, b):
  return jax.lax.dot(a, b)


def _elu(v):
  return jnp.where(v > 0, v, jnp.exp(jnp.minimum(v, 0.0)) - 1.0)


def _rownorm(v):
  ssq = _dot(v * v, jnp.ones((v.shape[1], 1), jnp.float32))
  return v * jax.lax.rsqrt(jnp.maximum(ssq, 1e-24))


ZR = RPT // 8
NBUF = 6
D = NHID


_SC_MESH = plsc.VectorSubcoreMesh(
    core_axis_name="c", subcore_axis_name="s", num_cores=NC, num_subcores=NS)
_SC_PARAMS = pltpu.CompilerParams(
    use_tc_tiling_on_sc=False, needs_layout_passes=False)


def _make_sc_agg(with_deg):

  def body(table, adj2d, *refs):
    if with_deg:
      out, dout, srcv, dstv, rows, zbuf, degp, acc, gsem, ssem = refs
    else:
      out, srcv, dstv, rows, zbuf, acc, gsem, ssem = refs
    c = lax.axis_index("c")
    s = lax.axis_index("s")
    wid = c * NS + s

    zeros = jnp.zeros((16,), jnp.float32)

    def zinit(r, carry):
      for k in range(D // 16):
        zbuf[r, pl.ds(k * 16, 16)] = zeros
      return carry

    lax.fori_loop(0, ZR, zinit, 0)
    for z in range(RPT // ZR):
      pltpu.sync_copy(zbuf, acc.at[pl.ds(s * RPT + z * ZR, ZR)])
    plsc.subcore_barrier()

    pltpu.sync_copy(adj2d.at[pl.ds(wid * NITER, NITER)], srcv)
    pltpu.sync_copy(adj2d.at[pl.ds(E // CH + wid * NITER, NITER)], dstv)

    for j in range(NBUF - 1):
      pltpu.async_copy(table.at[srcv.at[j]], rows.at[j], gsem.at[j])

    if with_deg:
      zeros16 = jnp.zeros((16,), jnp.float32)

      def dzinit(r, carry):
        degp[pl.ds(r * 16, 16)] = zeros16
        return carry

      lax.fori_loop(0, NP // 16, dzinit, 0)
      ones = jnp.full((16,), 1.0, jnp.float32)

      def dloop(i, carry):
        for k in range(CH // 16):
          plsc.addupdate_scatter(degp, [dstv[i, pl.ds(k * 16, 16)]], ones)
        return carry

      lax.fori_loop(0, NITER, dloop, 0)

    def eloop(i, carry):
      b = lax.rem(i, NBUF)
      pb = lax.rem(i + NBUF - 1, NBUF)
      g = i + NBUF - 1

      @pl.when(jnp.logical_and(i >= 1, g < NITER))
      def _():
        pltpu.make_async_copy(
            rows.at[pb], acc.at[dstv.at[i - 1]], ssem.at[pb]).wait()

      @pl.when(g < NITER)
      def _():
        pltpu.async_copy(table.at[srcv.at[g]], rows.at[pb], gsem.at[pb])

      pltpu.make_async_copy(table.at[srcv.at[i]], rows.at[b], gsem.at[b]).wait()
      pltpu.async_copy(rows.at[b], acc.at[dstv.at[i]], ssem.at[b], add=True)
      return carry

    lax.fori_loop(0, NITER, eloop, 0)
    for k in range(NBUF):
      ci = NITER - NBUF + k
      pltpu.make_async_copy(
          rows.at[ci % NBUF], acc.at[dstv.at[ci]], ssem.at[ci % NBUF]).wait()
    plsc.subcore_barrier()

    pltpu.sync_copy(acc.at[pl.ds(s * RPT, RPT)], out.at[c, pl.ds(s * RPT, RPT)])
    if with_deg:
      pltpu.sync_copy(degp, dout.at[wid])

  out_types = [jax.ShapeDtypeStruct((NC, NP, D), jnp.float32)]
  scratch = [
      pltpu.VMEM((NITER, CH), jnp.int32),
      pltpu.VMEM((NITER, CH), jnp.int32),
      pltpu.VMEM((NBUF, CH, D), jnp.float32),
      pltpu.VMEM((ZR, D), jnp.float32),
  ]
  if with_deg:
    out_types.append(jax.ShapeDtypeStruct((NW, NP), jnp.float32))
    scratch.append(pltpu.VMEM((NP,), jnp.float32))
  scratch += [
      pltpu.VMEM_SHARED((NP, D), jnp.float32),
      pltpu.SemaphoreType.DMA((NBUF,)),
      pltpu.SemaphoreType.DMA((NBUF,)),
  ]
  return pl.kernel(
      body,
      out_type=out_types,
      mesh=_SC_MESH,
      scratch_types=scratch,
      compiler_params=_SC_PARAMS,
  )


_sc_agg_deg = _make_sc_agg(True)
_sc_agg = _make_sc_agg(False)


RB = 2560
GRID = (N + RB - 1) // RB


def _ka1_body(x_ref, q_ref, Wl0x, Wl0q, T0_ref):
  T0_ref[...] = _dot(x_ref[...], Wl0x[...]) + _dot(q_ref[...], Wl0q[...])


def _ka2_body(t_ref, x_ref, q_ref, freq_ref, Wt1, bt1, Wt2, bt2,
              Wr0x, Wr0q, R0_ref, temb_ref):
  emb = (t_ref[...] * 4.0) * freq_ref[...]
  temb0 = jnp.concatenate([jnp.sin(emb), jnp.cos(emb)], axis=1)
  hmid = _elu(_dot(temb0, Wt1[...]) + bt1[...])
  temb_ref[...] = _dot(hmid, Wt2[...]) + bt2[...]
  R0_ref[...] = _dot(x_ref[...], Wr0x[...]) + _dot(q_ref[...], Wr0q[...])


def _kb_body(agg_ref, deg_ref, R0_ref, temb_ref, q_ref, bl0,
             Wl1x, Wl1q, Wr1x, Wr1q, T1_ref, R1_ref, rdeg_ref):
  a = agg_ref[0] + agg_ref[1]
  deg = jnp.sum(deg_ref[...], axis=0)
  rdeg = (1.0 / jnp.maximum(deg, 1.0)).reshape(RB, 1)
  out0 = a * rdeg + bl0[...] + R0_ref[...]
  h = jnp.maximum(_rownorm(out0) + temb_ref[...], 0.0)
  q = q_ref[...]
  T1_ref[...] = _dot(h, Wl1x[...]) + _dot(q, Wl1q[...])
  R1_ref[...] = _dot(h, Wr1x[...]) + _dot(q, Wr1q[...])
  rdeg_ref[...] = rdeg


def _kc_body(agg_ref, rdeg_ref, R1_ref, temb_ref, q_ref, bl1,
             Wf1h, Wf1q, bf1, Wf2, bf2, out_ref):
  a = agg_ref[0] + agg_ref[1]
  out1 = a * rdeg_ref[...] + bl1[...] + R1_ref[...]
  h = jnp.maximum(_rownorm(out1) + temb_ref[...], 0.0)
  q = q_ref[...]
  f = _elu(_dot(h, Wf1h[...]) + _dot(q, Wf1q[...]) + bf1[...])
  out_ref[...] = _dot(f, Wf2[...]) + bf2[...]


def _row_spec(d):
  return pl.BlockSpec((RB, d), lambda i: (i, 0))


def _full_spec(shape):
  nd = len(shape)
  return pl.BlockSpec(shape, lambda i, _n=nd: (0,) * _n)


def _agg_spec(d):
  return pl.BlockSpec((NC, RB, d), lambda i: (0, i, 0))


def kernel(x, q_Y_sample, adj, t, num_steps, W_t1, b_t1, W_t2, b_t2,
           Wl0, bl0, Wr0, Wl1, bl1, Wr1, Wf1, bf1, Wf2, bf2):
  del num_steps
  adj2d = adj.astype(jnp.int32).reshape(2 * (E // CH), CH)
  f32 = jnp.float32
  half = NHID // 2
  freq = jnp.exp(
      jnp.arange(half, dtype=f32) * (-math.log(10000.0) / (half - 1)))
  freq = freq.reshape(1, half)

  nfeat = x.shape[1]
  q = q_Y_sample

  ka1 = pl.pallas_call(
      _ka1_body,
      grid=(GRID,),
      in_specs=[
          _row_spec(nfeat), _row_spec(q.shape[1]),
          _full_spec((nfeat, NHID)), _full_spec((q.shape[1], NHID)),
      ],
      out_specs=[_row_spec(NHID)],
      out_shape=[jax.ShapeDtypeStruct((N, NHID), f32)],
  )
  (T0,) = ka1(x, q, Wl0[:nfeat], Wl0[nfeat:])

  agg0, degp = _sc_agg_deg(T0, adj2d)

  ka2 = pl.pallas_call(
      _ka2_body,
      grid=(GRID,),
      in_specs=[
          _row_spec(1), _row_spec(nfeat), _row_spec(q.shape[1]),
          _full_spec(freq.shape),
          _full_spec(W_t1.shape), _full_spec((1, b_t1.shape[0])),
          _full_spec(W_t2.shape), _full_spec((1, b_t2.shape[0])),
          _full_spec((nfeat, NHID)), _full_spec((q.shape[1], NHID)),
      ],
      out_specs=[_row_spec(NHID), _row_spec(NHID)],
      out_shape=[
          jax.ShapeDtypeStruct((N, NHID), f32),
          jax.ShapeDtypeStruct((N, NHID), f32),
      ],
  )
  R0, temb = ka2(
      t.reshape(N, 1), x, q, freq,
      W_t1, b_t1.reshape(1, -1), W_t2, b_t2.reshape(1, -1),
      Wr0[:nfeat], Wr0[nfeat:])

  kb = pl.pallas_call(
      _kb_body,
      grid=(GRID,),
      in_specs=[
          _agg_spec(NHID), pl.BlockSpec((NW, RB), lambda i: (0, i)),
          _row_spec(NHID), _row_spec(NHID),
          _row_spec(q.shape[1]), _full_spec((1, NHID)),
          _full_spec((NHID, NHID)), _full_spec((q.shape[1], NHID)),
          _full_spec((NHID, NHID)), _full_spec((q.shape[1], NHID)),
      ],
      out_specs=[_row_spec(NHID), _row_spec(NHID), _row_spec(1)],
      out_shape=[
          jax.ShapeDtypeStruct((N, NHID), f32),
          jax.ShapeDtypeStruct((N, NHID), f32),
          jax.ShapeDtypeStruct((N, 1), f32),
      ],
  )
  T1, R1, rdeg = kb(
      agg0, degp, R0, temb, q, bl0.reshape(1, -1),
      Wl1[:NHID], Wl1[NHID:], Wr1[:NHID], Wr1[NHID:])

  agg1, = _sc_agg(T1, adj2d)

  nout = bf2.shape[0]
  f1 = bf1.shape[0]
  kc = pl.pallas_call(
      _kc_body,
      grid=(GRID,),
      in_specs=[
          _agg_spec(NHID), _row_spec(1), _row_spec(NHID), _row_spec(NHID),
          _row_spec(q.shape[1]), _full_spec((1, NHID)),
          _full_spec((NHID, f1)), _full_spec((q.shape[1], f1)),
          _full_spec((1, f1)), _full_spec((f1, nout)), _full_spec((1, nout)),
      ],
      out_specs=[_row_spec(nout)],
      out_shape=[jax.ShapeDtypeStruct((N, nout), f32)],
  )
  (out,) = kc(
      agg1, rdeg, R1, temb, q, bl1.reshape(1, -1),
      Wf1[:NHID], Wf1[NHID:], bf1.reshape(1, -1), Wf2, bf2.reshape(1, -1))
  return out


# speedup vs baseline: 1.0192x; 1.0150x over previous
import math

import jax
import jax.numpy as jnp
from jax import lax
from jax.experimental import pallas as pl
from jax.experimental.pallas import tpu as pltpu
from jax.experimental.pallas import tpu_sc as plsc

N = 10000
E = 320000
NHID = 64

NC = 2
NS = 16
NW = NC * NS
EPT = E // NW
CH = 80
NITER = EPT // CH
RPT = 632
NP = NS * RPT

def _dot(a, b):
  return jax.lax.dot(a, b)


def _elu(v):
  return jnp.where(v > 0, v, jnp.exp(jnp.minimum(v, 0.0)) - 1.0)


def _rownorm(v):
  ssq = _dot(v * v, jnp.ones((v.shape[1], 1), jnp.float32))
  return v * jax.lax.rsqrt(jnp.maximum(ssq, 1e-24))


ZR = RPT // 8
NBUF = 8
D = NHID


_SC_MESH = plsc.VectorSubcoreMesh(
    core_axis_name="c", subcore_axis_name="s", num_cores=NC, num_subcores=NS)
_SC_PARAMS = pltpu.CompilerParams(
    use_tc_tiling_on_sc=False, needs_layout_passes=False)


def _make_sc_agg(with_deg):

  def body(table, adj2d, *refs):
    if with_deg:
      out, dout, srcv, dstv, rows, zbuf, degp, acc, gsem, ssem = refs
    else:
      out, srcv, dstv, rows, zbuf, acc, gsem, ssem = refs
    c = lax.axis_index("c")
    s = lax.axis_index("s")
    wid = c * NS + s

    zeros = jnp.zeros((16,), jnp.float32)

    def zinit(r, carry):
      for k in range(D // 16):
        zbuf[r, pl.ds(k * 16, 16)] = zeros
      return carry

    lax.fori_loop(0, ZR, zinit, 0)
    if with_deg:
      def dzinit(r, carry):
        degp[pl.ds(r * 16, 16)] = zeros
        return carry

      lax.fori_loop(0, NP // 16, dzinit, 0)
    for z in range(RPT // ZR):
      pltpu.sync_copy(zbuf, acc.at[pl.ds(s * RPT + z * ZR, ZR)])
    plsc.subcore_barrier()

    pltpu.sync_copy(adj2d.at[pl.ds(wid * NITER, NITER)], srcv)
    pltpu.sync_copy(adj2d.at[pl.ds(E // CH + wid * NITER, NITER)], dstv)

    ones = jnp.full((16,), 1.0, jnp.float32)
    for j in range(NBUF - 1):
      pltpu.async_copy(table.at[srcv.at[j]], rows.at[j], gsem.at[j])

    def eloop(i, carry):
      b = lax.rem(i, NBUF)
      pb = lax.rem(i + NBUF - 1, NBUF)
      g = i + NBUF - 1

      @pl.when(jnp.logical_and(i >= 1, g < NITER))
      def _():
        pltpu.make_async_copy(
            rows.at[pb], acc.at[dstv.at[i - 1]], ssem.at[pb]).wait()

      @pl.when(g < NITER)
      def _():
        pltpu.async_copy(table.at[srcv.at[g]], rows.at[pb], gsem.at[pb])

      if with_deg:
        for k in range(CH // 16):
          plsc.addupdate_scatter(degp, [dstv[i, pl.ds(k * 16, 16)]], ones)

      pltpu.make_async_copy(table.at[srcv.at[i]], rows.at[b], gsem.at[b]).wait()
      pltpu.async_copy(rows.at[b], acc.at[dstv.at[i]], ssem.at[b], add=True)
      return carry

    lax.fori_loop(0, NITER, eloop, 0)
    for k in range(NBUF):
      ci = NITER - NBUF + k
      pltpu.make_async_copy(
          rows.at[ci % NBUF], acc.at[dstv.at[ci]], ssem.at[ci % NBUF]).wait()
    plsc.subcore_barrier()

    pltpu.sync_copy(acc.at[pl.ds(s * RPT, RPT)], out.at[c, pl.ds(s * RPT, RPT)])
    if with_deg:
      pltpu.sync_copy(degp, dout.at[wid])

  out_types = [jax.ShapeDtypeStruct((NC, NP, D), jnp.float32)]
  scratch = [
      pltpu.VMEM((NITER, CH), jnp.int32),
      pltpu.VMEM((NITER, CH), jnp.int32),
      pltpu.VMEM((NBUF, CH, D), jnp.float32),
      pltpu.VMEM((ZR, D), jnp.float32),
  ]
  if with_deg:
    out_types.append(jax.ShapeDtypeStruct((NW, NP), jnp.float32))
    scratch.append(pltpu.VMEM((NP,), jnp.float32))
  scratch += [
      pltpu.VMEM_SHARED((NP, D), jnp.float32),
      pltpu.SemaphoreType.DMA((NBUF,)),
      pltpu.SemaphoreType.DMA((NBUF,)),
  ]
  return pl.kernel(
      body,
      out_type=out_types,
      mesh=_SC_MESH,
      scratch_types=scratch,
      compiler_params=_SC_PARAMS,
  )


_sc_agg_deg = _make_sc_agg(True)
_sc_agg = _make_sc_agg(False)


RB = 2560
GRID = (N + RB - 1) // RB


def _ka1_body(x_ref, q_ref, Wl0x, Wl0q, T0_ref):
  T0_ref[...] = _dot(x_ref[...], Wl0x[...]) + _dot(q_ref[...], Wl0q[...])


def _ka2_body(t_ref, x_ref, q_ref, freq_ref, Wt1, bt1, Wt2, bt2,
              Wr0x, Wr0q, R0_ref, temb_ref):
  emb = (t_ref[...] * 4.0) * freq_ref[...]
  temb0 = jnp.concatenate([jnp.sin(emb), jnp.cos(emb)], axis=1)
  hmid = _elu(_dot(temb0, Wt1[...]) + bt1[...])
  temb_ref[...] = _dot(hmid, Wt2[...]) + bt2[...]
  R0_ref[...] = _dot(x_ref[...], Wr0x[...]) + _dot(q_ref[...], Wr0q[...])


def _kb_body(agg_ref, deg_ref, R0_ref, temb_ref, q_ref, bl0,
             Wl1x, Wl1q, Wr1x, Wr1q, T1_ref, R1_ref, rdeg_ref):
  a = agg_ref[0] + agg_ref[1]
  deg = jnp.sum(deg_ref[...], axis=0)
  rdeg = (1.0 / jnp.maximum(deg, 1.0)).reshape(RB, 1)
  out0 = a * rdeg + bl0[...] + R0_ref[...]
  h = jnp.maximum(_rownorm(out0) + temb_ref[...], 0.0)
  q = q_ref[...]
  T1_ref[...] = _dot(h, Wl1x[...]) + _dot(q, Wl1q[...])
  R1_ref[...] = _dot(h, Wr1x[...]) + _dot(q, Wr1q[...])
  rdeg_ref[...] = rdeg


def _kc_body(agg_ref, rdeg_ref, R1_ref, temb_ref, q_ref, bl1,
             Wf1h, Wf1q, bf1, Wf2, bf2, out_ref):
  a = agg_ref[0] + agg_ref[1]
  out1 = a * rdeg_ref[...] + bl1[...] + R1_ref[...]
  h = jnp.maximum(_rownorm(out1) + temb_ref[...], 0.0)
  q = q_ref[...]
  f = _elu(_dot(h, Wf1h[...]) + _dot(q, Wf1q[...]) + bf1[...])
  out_ref[...] = _dot(f, Wf2[...]) + bf2[...]


def _row_spec(d):
  return pl.BlockSpec((RB, d), lambda i: (i, 0))


def _full_spec(shape):
  nd = len(shape)
  return pl.BlockSpec(shape, lambda i, _n=nd: (0,) * _n)


def _agg_spec(d):
  return pl.BlockSpec((NC, RB, d), lambda i: (0, i, 0))


def kernel(x, q_Y_sample, adj, t, num_steps, W_t1, b_t1, W_t2, b_t2,
           Wl0, bl0, Wr0, Wl1, bl1, Wr1, Wf1, bf1, Wf2, bf2):
  del num_steps
  adj2d = adj.astype(jnp.int32).reshape(2 * (E // CH), CH)
  f32 = jnp.float32
  half = NHID // 2
  freq = jnp.exp(
      jnp.arange(half, dtype=f32) * (-math.log(10000.0) / (half - 1)))
  freq = freq.reshape(1, half)

  nfeat = x.shape[1]
  q = q_Y_sample

  ka1 = pl.pallas_call(
      _ka1_body,
      grid=(GRID,),
      in_specs=[
          _row_spec(nfeat), _row_spec(q.shape[1]),
          _full_spec((nfeat, NHID)), _full_spec((q.shape[1], NHID)),
      ],
      out_specs=[_row_spec(NHID)],
      out_shape=[jax.ShapeDtypeStruct((N, NHID), f32)],
  )
  (T0,) = ka1(x, q, Wl0[:nfeat], Wl0[nfeat:])

  agg0, degp = _sc_agg_deg(T0, adj2d)

  ka2 = pl.pallas_call(
      _ka2_body,
      grid=(GRID,),
      in_specs=[
          _row_spec(1), _row_spec(nfeat), _row_spec(q.shape[1]),
          _full_spec(freq.shape),
          _full_spec(W_t1.shape), _full_spec((1, b_t1.shape[0])),
          _full_spec(W_t2.shape), _full_spec((1, b_t2.shape[0])),
          _full_spec((nfeat, NHID)), _full_spec((q.shape[1], NHID)),
      ],
      out_specs=[_row_spec(NHID), _row_spec(NHID)],
      out_shape=[
          jax.ShapeDtypeStruct((N, NHID), f32),
          jax.ShapeDtypeStruct((N, NHID), f32),
      ],
  )
  R0, temb = ka2(
      t.reshape(N, 1), x, q, freq,
      W_t1, b_t1.reshape(1, -1), W_t2, b_t2.reshape(1, -1),
      Wr0[:nfeat], Wr0[nfeat:])

  kb = pl.pallas_call(
      _kb_body,
      grid=(GRID,),
      in_specs=[
          _agg_spec(NHID), pl.BlockSpec((NW, RB), lambda i: (0, i)),
          _row_spec(NHID), _row_spec(NHID),
          _row_spec(q.shape[1]), _full_spec((1, NHID)),
          _full_spec((NHID, NHID)), _full_spec((q.shape[1], NHID)),
          _full_spec((NHID, NHID)), _full_spec((q.shape[1], NHID)),
      ],
      out_specs=[_row_spec(NHID), _row_spec(NHID), _row_spec(1)],
      out_shape=[
          jax.ShapeDtypeStruct((N, NHID), f32),
          jax.ShapeDtypeStruct((N, NHID), f32),
          jax.ShapeDtypeStruct((N, 1), f32),
      ],
  )
  T1, R1, rdeg = kb(
      agg0, degp, R0, temb, q, bl0.reshape(1, -1),
      Wl1[:NHID], Wl1[NHID:], Wr1[:NHID], Wr1[NHID:])

  agg1, = _sc_agg(T1, adj2d)

  nout = bf2.shape[0]
  f1 = bf1.shape[0]
  kc = pl.pallas_call(
      _kc_body,
      grid=(GRID,),
      in_specs=[
          _agg_spec(NHID), _row_spec(1), _row_spec(NHID), _row_spec(NHID),
          _row_spec(q.shape[1]), _full_spec((1, NHID)),
          _full_spec((NHID, f1)), _full_spec((q.shape[1], f1)),
          _full_spec((1, f1)), _full_spec((f1, nout)), _full_spec((1, nout)),
      ],
      out_specs=[_row_spec(nout)],
      out_shape=[jax.ShapeDtypeStruct((N, nout), f32)],
  )
  (out,) = kc(
      agg1, rdeg, R1, temb, q, bl1.reshape(1, -1),
      Wf1[:NHID], Wf1[NHID:], bf1.reshape(1, -1), Wf2, bf2.reshape(1, -1))
  return out


# speedup vs baseline: 1.0503x; 1.0305x over previous
import math

import jax
import jax.numpy as jnp
from jax import lax
from jax.experimental import pallas as pl
from jax.experimental.pallas import tpu as pltpu
from jax.experimental.pallas import tpu_sc as plsc

N = 10000
E = 320000
NHID = 64

NC = 2
NS = 16
NW = NC * NS
EPT = E // NW
CH = 80
NITER = EPT // CH
RPT = 632
NP = NS * RPT

def _dot(a, b):
  return jax.lax.dot(a, b)


def _dotT(at, b):
  return jax.lax.dot_general(at, b, (((0,), (0,)), ((), ())))


def _elu(v):
  return jnp.where(v > 0, v, jnp.exp(jnp.minimum(v, 0.0)) - 1.0)


def _rownorm(v):
  ssq = _dot(v * v, jnp.ones((v.shape[1], 1), jnp.float32))
  return v * jax.lax.rsqrt(jnp.maximum(ssq, 1e-24))


ZR = RPT // 8
NBUF = 8
D = NHID


_SC_MESH = plsc.VectorSubcoreMesh(
    core_axis_name="c", subcore_axis_name="s", num_cores=NC, num_subcores=NS)
_SC_PARAMS = pltpu.CompilerParams(
    use_tc_tiling_on_sc=False, needs_layout_passes=False)


def _make_sc_agg(with_deg):

  def body(table, adj2d, *refs):
    if with_deg:
      out, dout, srcv, dstv, rows, zbuf, degp, acc, gsem, ssem = refs
    else:
      out, srcv, dstv, rows, zbuf, acc, gsem, ssem = refs
    c = lax.axis_index("c")
    s = lax.axis_index("s")
    wid = c * NS + s

    zeros = jnp.zeros((16,), jnp.float32)

    def zinit(r, carry):
      for k in range(D // 16):
        zbuf[r, pl.ds(k * 16, 16)] = zeros
      return carry

    lax.fori_loop(0, ZR, zinit, 0)
    if with_deg:
      def dzinit(r, carry):
        degp[pl.ds(r * 16, 16)] = zeros
        return carry

      lax.fori_loop(0, NP // 16, dzinit, 0)
    for z in range(RPT // ZR):
      pltpu.sync_copy(zbuf, acc.at[pl.ds(s * RPT + z * ZR, ZR)])
    plsc.subcore_barrier()

    pltpu.sync_copy(adj2d.at[pl.ds(wid * NITER, NITER)], srcv)
    pltpu.sync_copy(adj2d.at[pl.ds(E // CH + wid * NITER, NITER)], dstv)

    ones = jnp.full((16,), 1.0, jnp.float32)
    for j in range(NBUF - 1):
      pltpu.async_copy(table.at[srcv.at[j]], rows.at[j], gsem.at[j])

    def eloop(i, carry):
      b = lax.rem(i, NBUF)
      pb = lax.rem(i + NBUF - 1, NBUF)
      g = i + NBUF - 1

      @pl.when(jnp.logical_and(i >= 1, g < NITER))
      def _():
        pltpu.make_async_copy(
            rows.at[pb], acc.at[dstv.at[i - 1]], ssem.at[pb]).wait()

      @pl.when(g < NITER)
      def _():
        pltpu.async_copy(table.at[srcv.at[g]], rows.at[pb], gsem.at[pb])

      if with_deg:
        for k in range(CH // 16):
          plsc.addupdate_scatter(degp, [dstv[i, pl.ds(k * 16, 16)]], ones)

      pltpu.make_async_copy(table.at[srcv.at[i]], rows.at[b], gsem.at[b]).wait()
      pltpu.async_copy(rows.at[b], acc.at[dstv.at[i]], ssem.at[b], add=True)
      return carry

    lax.fori_loop(0, NITER, eloop, 0)
    for k in range(NBUF):
      ci = NITER - NBUF + k
      pltpu.make_async_copy(
          rows.at[ci % NBUF], acc.at[dstv.at[ci]], ssem.at[ci % NBUF]).wait()
    plsc.subcore_barrier()

    pltpu.sync_copy(acc.at[pl.ds(s * RPT, RPT)], out.at[c, pl.ds(s * RPT, RPT)])
    if with_deg:
      pltpu.sync_copy(degp, dout.at[wid])

  out_types = [jax.ShapeDtypeStruct((NC, NP, D), jnp.float32)]
  scratch = [
      pltpu.VMEM((NITER, CH), jnp.int32),
      pltpu.VMEM((NITER, CH), jnp.int32),
      pltpu.VMEM((NBUF, CH, D), jnp.float32),
      pltpu.VMEM((ZR, D), jnp.float32),
  ]
  if with_deg:
    out_types.append(jax.ShapeDtypeStruct((NW, NP), jnp.float32))
    scratch.append(pltpu.VMEM((NP,), jnp.float32))
  scratch += [
      pltpu.VMEM_SHARED((NP, D), jnp.float32),
      pltpu.SemaphoreType.DMA((NBUF,)),
      pltpu.SemaphoreType.DMA((NBUF,)),
  ]
  return pl.kernel(
      body,
      out_type=out_types,
      mesh=_SC_MESH,
      scratch_types=scratch,
      compiler_params=_SC_PARAMS,
  )


_sc_agg_deg = _make_sc_agg(True)
_sc_agg = _make_sc_agg(False)


RB = 2560
GRID = (N + RB - 1) // RB


def _ka1_body(x_ref, qt_ref, Wl0x, Wl0q, T0_ref):
  T0_ref[...] = _dot(x_ref[...], Wl0x[...]) + _dotT(qt_ref[...], Wl0q[...])


def _ka2_body(t_ref, x_ref, qt_ref, freq_ref, Wt1, bt1, Wt2, bt2,
              Wr0x, Wr0q, R0_ref, temb_ref):
  emb = (t_ref[...] * 4.0) * freq_ref[...]
  temb0 = jnp.concatenate([jnp.sin(emb), jnp.cos(emb)], axis=1)
  hmid = _elu(_dot(temb0, Wt1[...]) + bt1[...])
  temb_ref[...] = _dot(hmid, Wt2[...]) + bt2[...]
  R0_ref[...] = _dot(x_ref[...], Wr0x[...]) + _dotT(qt_ref[...], Wr0q[...])


def _kb_body(agg_ref, deg_ref, R0_ref, temb_ref, qt_ref, bl0,
             Wl1x, Wl1q, Wr1x, Wr1q, T1_ref, R1_ref, rdeg_ref):
  a = agg_ref[0] + agg_ref[1]
  deg = jnp.sum(deg_ref[...], axis=0)
  rdeg = (1.0 / jnp.maximum(deg, 1.0)).reshape(RB, 1)
  out0 = a * rdeg + bl0[...] + R0_ref[...]
  h = jnp.maximum(_rownorm(out0) + temb_ref[...], 0.0)
  qt = qt_ref[...]
  T1_ref[...] = _dot(h, Wl1x[...]) + _dotT(qt, Wl1q[...])
  R1_ref[...] = _dot(h, Wr1x[...]) + _dotT(qt, Wr1q[...])
  rdeg_ref[...] = rdeg


def _kc_body(agg_ref, rdeg_ref, R1_ref, temb_ref, qt_ref, bl1,
             Wf1h, Wf1q, bf1, Wf2, bf2, out_ref):
  a = agg_ref[0] + agg_ref[1]
  out1 = a * rdeg_ref[...] + bl1[...] + R1_ref[...]
  h = jnp.maximum(_rownorm(out1) + temb_ref[...], 0.0)
  f = _elu(_dot(h, Wf1h[...]) + _dotT(qt_ref[...], Wf1q[...]) + bf1[...])
  out_ref[...] = _dot(f, Wf2[...]) + bf2[...]


def _row_spec(d):
  return pl.BlockSpec((RB, d), lambda i: (i, 0))


def _full_spec(shape):
  nd = len(shape)
  return pl.BlockSpec(shape, lambda i, _n=nd: (0,) * _n)


def _agg_spec(d):
  return pl.BlockSpec((NC, RB, d), lambda i: (0, i, 0))


def kernel(x, q_Y_sample, adj, t, num_steps, W_t1, b_t1, W_t2, b_t2,
           Wl0, bl0, Wr0, Wl1, bl1, Wr1, Wf1, bf1, Wf2, bf2):
  del num_steps
  adj2d = adj.astype(jnp.int32).reshape(2 * (E // CH), CH)
  f32 = jnp.float32
  half = NHID // 2
  freq = jnp.exp(
      jnp.arange(half, dtype=f32) * (-math.log(10000.0) / (half - 1)))
  freq = freq.reshape(1, half)

  nfeat = x.shape[1]
  q = q_Y_sample
  nlbl = q.shape[1]
  qt = q.T
  _qt_spec = pl.BlockSpec((nlbl, RB), lambda i: (0, i))

  ka1 = pl.pallas_call(
      _ka1_body,
      grid=(GRID,),
      in_specs=[
          _row_spec(nfeat), _qt_spec,
          _full_spec((nfeat, NHID)), _full_spec((nlbl, NHID)),
      ],
      out_specs=[_row_spec(NHID)],
      out_shape=[jax.ShapeDtypeStruct((N, NHID), f32)],
  )
  (T0,) = ka1(x, qt, Wl0[:nfeat], Wl0[nfeat:])

  agg0, degp = _sc_agg_deg(T0, adj2d)

  ka2 = pl.pallas_call(
      _ka2_body,
      grid=(GRID,),
      in_specs=[
          _row_spec(1), _row_spec(nfeat), _qt_spec,
          _full_spec(freq.shape),
          _full_spec(W_t1.shape), _full_spec((1, b_t1.shape[0])),
          _full_spec(W_t2.shape), _full_spec((1, b_t2.shape[0])),
          _full_spec((nfeat, NHID)), _full_spec((nlbl, NHID)),
      ],
      out_specs=[_row_spec(NHID), _row_spec(NHID)],
      out_shape=[
          jax.ShapeDtypeStruct((N, NHID), f32),
          jax.ShapeDtypeStruct((N, NHID), f32),
      ],
  )
  R0, temb = ka2(
      t.reshape(N, 1), x, qt, freq,
      W_t1, b_t1.reshape(1, -1), W_t2, b_t2.reshape(1, -1),
      Wr0[:nfeat], Wr0[nfeat:])

  kb = pl.pallas_call(
      _kb_body,
      grid=(GRID,),
      in_specs=[
          _agg_spec(NHID), pl.BlockSpec((NW, RB), lambda i: (0, i)),
          _row_spec(NHID), _row_spec(NHID),
          _qt_spec, _full_spec((1, NHID)),
          _full_spec((NHID, NHID)), _full_spec((nlbl, NHID)),
          _full_spec((NHID, NHID)), _full_spec((nlbl, NHID)),
      ],
      out_specs=[_row_spec(NHID), _row_spec(NHID), _row_spec(1)],
      out_shape=[
          jax.ShapeDtypeStruct((N, NHID), f32),
          jax.ShapeDtypeStruct((N, NHID), f32),
          jax.ShapeDtypeStruct((N, 1), f32),
      ],
  )
  T1, R1, rdeg = kb(
      agg0, degp, R0, temb, qt, bl0.reshape(1, -1),
      Wl1[:NHID], Wl1[NHID:], Wr1[:NHID], Wr1[NHID:])

  agg1, = _sc_agg(T1, adj2d)

  nout = bf2.shape[0]
  f1 = bf1.shape[0]
  kc = pl.pallas_call(
      _kc_body,
      grid=(GRID,),
      in_specs=[
          _agg_spec(NHID), _row_spec(1), _row_spec(NHID), _row_spec(NHID),
          _qt_spec, _full_spec((1, NHID)),
          _full_spec((NHID, f1)), _full_spec((nlbl, f1)),
          _full_spec((1, f1)), _full_spec((f1, nout)), _full_spec((1, nout)),
      ],
      out_specs=[_row_spec(nout)],
      out_shape=[jax.ShapeDtypeStruct((N, nout), f32)],
  )
  (out,) = kc(
      agg1, rdeg, R1, temb, qt, bl1.reshape(1, -1),
      Wf1[:NHID], Wf1[NHID:], bf1.reshape(1, -1), Wf2, bf2.reshape(1, -1))
  return out
